# Initial kernel scaffold; baseline (speedup 1.0000x reference)
import jax
import jax.numpy as jnp
from jax.experimental import pallas as pl


def kernel(one_hot, features, gemme_features, a_res, W1, b1, W2, b2, W3, b3, W4, b4, W5, b5, L1w, L1b, L2w, L2b, L3w, L3b):
    raise NotImplementedError("write your pallas kernel here")


# speedup vs baseline: 11.3066x; 11.3066x over previous
import functools

import jax
import jax.numpy as jnp
from jax import lax
from jax.experimental import pallas as pl
from jax.experimental.pallas import tpu as pltpu
from jax.experimental.pallas import tpu_sc as plsc

_N = 50000
_E = 800000
_NC = 2
_NS = 16
_CHUNK = 128
_BLK = 14
_NBLK = 14
_NCH = _BLK * _NBLK
_E_PAD = _NC * _NS * _NCH * _CHUNK
_N_PAD = 50048
_RPW = _N_PAD // _NS
_CB = 184


def _sc_segment_sum(h, src3, dst3, zeros, d):
    mesh = plsc.VectorSubcoreMesh(core_axis_name="c", subcore_axis_name="s")

    @functools.partial(
        pl.kernel,
        out_type=jax.ShapeDtypeStruct((_NC, _N_PAD, d), jnp.float32),
        mesh=mesh,
        scratch_types=[
            pltpu.VMEM((_BLK, _CHUNK), jnp.int32),
            pltpu.VMEM((_BLK, _CHUNK), jnp.int32),
            pltpu.VMEM((_CHUNK, d), jnp.float32),
            pltpu.VMEM((_CB, d), jnp.float32),
            pltpu.VMEM_SHARED((_N_PAD, d), jnp.float32),
            pltpu.SemaphoreType.DMA,
        ],
        compiler_params=pltpu.CompilerParams(use_tc_tiling_on_sc=False),
    )
    def k(h_hbm, src_hbm, dst_hbm, z_hbm, out_hbm, src_v, dst_v, rows_v,
          cbuf, agg, sem):
        c = lax.axis_index("c")
        s = lax.axis_index("s")
        base = s * _RPW

        pltpu.sync_copy(z_hbm, cbuf)

        def zbody(j, carry):
            pltpu.sync_copy(cbuf, agg.at[pl.ds(base + j * _CB, _CB)])
            return carry

        lax.fori_loop(0, _RPW // _CB, zbody, 0)
        plsc.subcore_barrier()

        def blk_body(b, carry):
            pltpu.sync_copy(src_hbm.at[c, s, pl.ds(b * _BLK, _BLK)], src_v)
            pltpu.sync_copy(dst_hbm.at[c, s, pl.ds(b * _BLK, _BLK)], dst_v)

            def ebody(j, carry2):
                pltpu.async_copy(h_hbm.at[src_v.at[j]], rows_v, sem).wait()
                pltpu.sync_copy(rows_v, agg.at[dst_v.at[j]], add=True)
                return carry2

            lax.fori_loop(0, _BLK, ebody, 0)
            return carry

        lax.fori_loop(0, _NBLK, blk_body, 0)
        plsc.subcore_barrier()

        def obody(j, carry):
            off = base + j * _CB
            pltpu.sync_copy(agg.at[pl.ds(off, _CB)], cbuf)
            pltpu.sync_copy(cbuf, out_hbm.at[c, pl.ds(off, _CB)])
            return carry

        lax.fori_loop(0, _RPW // _CB, obody, 0)

    return k(h, src3, dst3, zeros)


def _elu(x):
    return jnp.where(x > 0, x, jnp.exp(jnp.minimum(x, 0.0)) - 1.0)


def _tc_first_mm(one_hot, features, W1p):
    bm = 2000
    n_oh = one_hot.shape[1]

    def body(oh_ref, ft_ref, w_ref, o_ref):
        w = w_ref[...]
        o_ref[...] = (
            jnp.dot(oh_ref[...], w[:n_oh], preferred_element_type=jnp.float32)
            + jnp.dot(ft_ref[...], w[n_oh:], preferred_element_type=jnp.float32)
        )

    dout = W1p.shape[1]
    return pl.pallas_call(
        body,
        grid=(_N // bm,),
        in_specs=[
            pl.BlockSpec((bm, one_hot.shape[1]), lambda i: (i, 0)),
            pl.BlockSpec((bm, features.shape[1]), lambda i: (i, 0)),
            pl.BlockSpec(W1p.shape, lambda i: (0, 0)),
        ],
        out_specs=pl.BlockSpec((bm, dout), lambda i: (i, 0)),
        out_shape=jax.ShapeDtypeStruct((_N, dout), jnp.float32),
    )(one_hot, features, W1p)


def _tc_merge_mm(p, b2d, W):
    bm = 3128
    din, dout = W.shape

    def body(p0_ref, p1_ref, b_ref, w_ref, o_ref):
        y = _elu(p0_ref[0] + p1_ref[0] + b_ref[...])
        o_ref[...] = jnp.dot(y, w_ref[...], preferred_element_type=jnp.float32)

    return pl.pallas_call(
        body,
        grid=(_N_PAD // bm,),
        in_specs=[
            pl.BlockSpec((1, bm, din), lambda i: (0, i, 0)),
            pl.BlockSpec((1, bm, din), lambda i: (1, i, 0)),
            pl.BlockSpec((1, din), lambda i: (0, 0)),
            pl.BlockSpec((din, dout), lambda i: (0, 0)),
        ],
        out_specs=pl.BlockSpec((bm, dout), lambda i: (i, 0)),
        out_shape=jax.ShapeDtypeStruct((_N_PAD, dout), jnp.float32),
    )(p, p, b2d, W)


def _tc_head(p, b5, L1w, L1b, L2w, L2b, L3w, L3b):
    bm = 3128

    def body(p0_ref, p1_ref, b5_ref, w1_ref, b1_ref, w2_ref, b2_ref,
             w3_ref, b3_ref, o_ref):
        y = _elu(p0_ref[0] + p1_ref[0] + b5_ref[...])
        z = _elu(
            jnp.dot(y, w1_ref[...], preferred_element_type=jnp.float32)
            + b1_ref[...])
        z = _elu(
            jnp.dot(z, w2_ref[...], preferred_element_type=jnp.float32)
            + b2_ref[...])
        o_ref[...] = jax.nn.sigmoid(
            jnp.dot(z, w3_ref[...], preferred_element_type=jnp.float32)
            + b3_ref[...])

    din = p.shape[2]
    return pl.pallas_call(
        body,
        grid=(_N_PAD // bm,),
        in_specs=[
            pl.BlockSpec((1, bm, din), lambda i: (0, i, 0)),
            pl.BlockSpec((1, bm, din), lambda i: (1, i, 0)),
            pl.BlockSpec((1, din), lambda i: (0, 0)),
            pl.BlockSpec(L1w.shape, lambda i: (0, 0)),
            pl.BlockSpec((1, L1w.shape[1]), lambda i: (0, 0)),
            pl.BlockSpec(L2w.shape, lambda i: (0, 0)),
            pl.BlockSpec((1, L2w.shape[1]), lambda i: (0, 0)),
            pl.BlockSpec(L3w.shape, lambda i: (0, 0)),
            pl.BlockSpec((1, L3w.shape[1]), lambda i: (0, 0)),
        ],
        out_specs=pl.BlockSpec((bm, 1), lambda i: (i, 0)),
        out_shape=jax.ShapeDtypeStruct((_N_PAD, 1), jnp.float32),
    )(p, p, b5, L1w, L1b, L2w, L2b, L3w, L3b)


def _pad_cols(W, dout):
    return jnp.pad(W, ((0, 0), (0, dout - W.shape[1])))


def _pad_rows(W, din):
    return jnp.pad(W, ((0, din - W.shape[0]), (0, 0)))


def _pad_vec(b, d):
    return jnp.pad(b, (0, d - b.shape[0])).reshape(1, -1)


def kernel(one_hot, features, gemme_features, a_res,
           W1, b1, W2, b2, W3, b3, W4, b4, W5, b5,
           L1w, L1b, L2w, L2b, L3w, L3b):
    src = a_res[0].astype(jnp.int32)
    dst = a_res[1].astype(jnp.int32)
    pad = _E_PAD - _E
    src3 = jnp.concatenate([src, jnp.zeros((pad,), jnp.int32)])
    src3 = src3.reshape(_NC, _NS, _NCH, _CHUNK)
    dst3 = jnp.concatenate([dst, jnp.full((pad,), _N, jnp.int32)])
    dst3 = dst3.reshape(_NC, _NS, _NCH, _CHUNK)

    dpad = [32, 16, 16, 8, 8]
    Ws = [_pad_cols(W1, 32),
          _pad_cols(_pad_rows(W2, 32), 16),
          _pad_cols(W3, 16),
          _pad_rows(W4, 16),
          _pad_cols(W5, 8)]
    bs = [_pad_vec(b1, 32), _pad_vec(b2, 16), _pad_vec(b3, 16),
          _pad_vec(b4, 8), _pad_vec(b5, 8)]

    h = _tc_first_mm(one_hot, features, Ws[0])
    for i in range(1, 5):
        d = dpad[i - 1]
        p = _sc_segment_sum(h, src3, dst3, jnp.zeros((_CB, d), jnp.float32), d)
        h = _tc_merge_mm(p, bs[i - 1], Ws[i])
    d = dpad[4]
    p = _sc_segment_sum(h, src3, dst3, jnp.zeros((_CB, d), jnp.float32), d)
    out = _tc_head(p, bs[4], _pad_rows(L1w, 8), L1b.reshape(1, -1),
                   L2w, L2b.reshape(1, -1), L3w, L3b.reshape(1, -1))
    return out[:_N]
